# Initial kernel scaffold; baseline (speedup 1.0000x reference)
import jax
import jax.numpy as jnp
from jax.experimental import pallas as pl


def kernel(user_table, item_table, edge_index, users, items):
    raise NotImplementedError("write your pallas kernel here")


# speedup vs baseline: 1.0009x; 1.0009x over previous
import jax
import jax.numpy as jnp
from jax.experimental import pallas as pl

_NUM_USERS = 100000
_NUM_ITEMS = 100000
_EMBED = 64
_LAYERS = 3
_B = 4096
_BLK = 512
_NSTEPS = _B // _BLK


def _loss_kernel(ub_ref, u_ref, ib_ref, i_ref, out_ref):
    step = pl.program_id(0)

    @pl.when(step == 0)
    def _init():
        out_ref[...] = jnp.zeros_like(out_ref)

    u = u_ref[...]
    it = i_ref[...]
    un = u / jnp.maximum(jnp.sqrt(jnp.sum(u * u, axis=1, keepdims=True)), 1e-12)
    itn = it / jnp.maximum(jnp.sqrt(jnp.sum(it * it, axis=1, keepdims=True)), 1e-12)

    ubr = ub_ref[...]
    ibr = ib_ref[...]
    ub = ubr / jnp.maximum(jnp.sqrt(jnp.sum(ubr * ubr, axis=1, keepdims=True)), 1e-12)
    ib = ibr / jnp.maximum(jnp.sqrt(jnp.sum(ibr * ibr, axis=1, keepdims=True)), 1e-12)

    align_p = jnp.sum((ub - ib) ** 2)

    squ = jnp.sum(un * un, axis=1)
    sqi = jnp.sum(itn * itn, axis=1)
    squ_b = jnp.sum(ub * ub, axis=1)
    sqi_b = jnp.sum(ib * ib, axis=1)

    gu = jnp.dot(ub, un.T, preferred_element_type=jnp.float32)
    du = jnp.maximum(squ_b[:, None] + squ[None, :] - 2.0 * gu, 0.0)
    su_p = jnp.sum(jnp.exp(-2.0 * du))

    gi = jnp.dot(ib, itn.T, preferred_element_type=jnp.float32)
    di = jnp.maximum(sqi_b[:, None] + sqi[None, :] - 2.0 * gi, 0.0)
    si_p = jnp.sum(jnp.exp(-2.0 * di))

    lane = jax.lax.broadcasted_iota(jnp.int32, (1, 128), 1)
    vec = (jnp.where(lane == 0, align_p, 0.0)
           + jnp.where(lane == 1, su_p, 0.0)
           + jnp.where(lane == 2, si_p, 0.0))
    out_ref[...] += vec


def _losses(users_emb, items_emb):
    out = pl.pallas_call(
        _loss_kernel,
        grid=(_NSTEPS,),
        in_specs=[
            pl.BlockSpec((_BLK, _EMBED), lambda i: (i, 0)),
            pl.BlockSpec((_B, _EMBED), lambda i: (0, 0)),
            pl.BlockSpec((_BLK, _EMBED), lambda i: (i, 0)),
            pl.BlockSpec((_B, _EMBED), lambda i: (0, 0)),
        ],
        out_specs=pl.BlockSpec((1, 128), lambda i: (0, 0)),
        out_shape=jax.ShapeDtypeStruct((1, 128), jnp.float32),
    )(users_emb, users_emb, items_emb, items_emb)
    align_loss = out[0, 0] / _B
    cnt = _B * (_B - 1) / 2.0
    unif_u = jnp.log(((out[0, 1] - _B) / 2.0) / cnt)
    unif_i = jnp.log(((out[0, 2] - _B) / 2.0) / cnt)
    return align_loss, (unif_u + unif_i) / 2.0


def kernel(user_table, item_table, edge_index, users, items):
    n = _NUM_USERS + _NUM_ITEMS
    src_u = edge_index[0]
    dst_i = edge_index[1] + _NUM_USERS
    s = jnp.concatenate([src_u, dst_i])
    d = jnp.concatenate([dst_i, src_u])
    deg = jax.ops.segment_sum(jnp.ones_like(s, dtype=jnp.float32), s, num_segments=n)
    d_inv = jnp.where(deg > 0, deg ** -0.5, 0.0)
    w = (d_inv[s] * d_inv[d])[:, None]

    emb = jnp.concatenate([user_table, item_table], axis=0)
    acc = emb
    cur = emb
    for _ in range(_LAYERS):
        cur = jax.ops.segment_sum(w * cur[s], d, num_segments=n)
        acc = acc + cur
    light = acc * (1.0 / (_LAYERS + 1))

    users_emb = light[users]
    items_emb = light[items + _NUM_USERS]
    return _losses(users_emb, items_emb)
